# Initial kernel scaffold; baseline (speedup 1.0000x reference)
import jax
import jax.numpy as jnp
from jax.experimental import pallas as pl


def kernel(x, edge_index, Wl0, Wr0, att0, b0, g0, be0, Wl1, Wr1, att1, b1, g1, be1, Wc1, bc1, Wc2, bc2):
    raise NotImplementedError("write your pallas kernel here")


# speedup vs baseline: 45.6916x; 45.6916x over previous
import functools

import jax
import jax.numpy as jnp
from jax import lax
from jax.experimental import pallas as pl
from jax.experimental.pallas import tpu as pltpu
from jax.experimental.pallas import tpu_sc as plsc

_NC = 2
_NS = 16
_L = 16
_CH = 80


def _edge_pass(xl, xr, src, dst, attf, H, C):
    N = xl.shape[0]
    E = src.shape[0]
    D = H * C
    NW = _NC * _NS
    EPW = E // NW
    NCH = EPW // _CH
    RPT = N // _NS
    assert EPW * NW == E and NCH * _CH == EPW and RPT * _NS == N

    zrow = jnp.zeros((RPT, D), jnp.float32)
    zden = jnp.zeros((RPT, _L), jnp.float32)

    mesh = plsc.VectorSubcoreMesh(core_axis_name="c", subcore_axis_name="s")

    @functools.partial(
        pl.kernel,
        out_type=(
            jax.ShapeDtypeStruct((_NC, N, D), jnp.float32),
            jax.ShapeDtypeStruct((_NC, N, _L), jnp.float32),
        ),
        mesh=mesh,
        compiler_params=pltpu.CompilerParams(
            needs_layout_passes=False, use_tc_tiling_on_sc=False),
        scratch_types=[
            pltpu.VMEM((D,), jnp.float32),
            pltpu.VMEM((_CH,), jnp.int32),
            pltpu.VMEM((_CH,), jnp.int32),
            pltpu.VMEM((_CH, D), jnp.float32),
            pltpu.VMEM((_CH, D), jnp.float32),
            pltpu.VMEM((_CH, _L), jnp.float32),
            pltpu.VMEM_SHARED((N, D), jnp.float32),
            pltpu.VMEM_SHARED((N, _L), jnp.float32),
            pltpu.SemaphoreType.DMA,
        ],
    )
    def k(xl_hbm, xr_hbm, src_hbm, dst_hbm, att_hbm, zrow_hbm, zden_hbm,
          nump_hbm, denp_hbm,
          att_v, si_v, di_v, xlr_v, xrr_v, pp_v, num_sh, den_sh, sem):
        c = lax.axis_index("c")
        s = lax.axis_index("s")
        wid = s * _NC + c

        pltpu.sync_copy(att_hbm, att_v)
        pltpu.sync_copy(zrow_hbm, num_sh.at[pl.ds(s * RPT, RPT)])
        pltpu.sync_copy(zden_hbm, den_sh.at[pl.ds(s * RPT, RPT)])

        def zi(i, cy):
            pp_v[i, pl.ds(0, _L)] = jnp.zeros((_L,), jnp.float32)
            return cy

        lax.fori_loop(0, _CH, zi, 0)
        plsc.subcore_barrier()

        lane = lax.iota(jnp.int32, _L)
        lmask = [lane == i for i in range(_L)]
        base0 = wid * EPW

        def chunk(kk, carry):
            base = base0 + kk * _CH
            pltpu.sync_copy(src_hbm.at[pl.ds(base, _CH)], si_v)
            pltpu.sync_copy(dst_hbm.at[pl.ds(base, _CH)], di_v)
            cp1 = pltpu.async_copy(xl_hbm.at[si_v], xlr_v, sem)
            cp2 = pltpu.async_copy(xr_hbm.at[di_v], xrr_v, sem)
            cp1.wait()
            cp2.wait()

            def grp(j, cy):
                avec = [jnp.zeros((_L,), jnp.float32) for _ in range(H)]
                for i16 in range(_L):
                    i = j * _L + i16
                    for h in range(H):
                        acc = None
                        for g in range(C // _L):
                            col = h * C + g * _L
                            v = (xlr_v[i, pl.ds(col, _L)]
                                 + xrr_v[i, pl.ds(col, _L)])
                            t = jnp.maximum(v, 0.2 * v)
                            term = t * att_v[pl.ds(col, _L)]
                            acc = term if acc is None else acc + term
                        avec[h] = jnp.where(lmask[i16], jnp.sum(acc), avec[h])
                pvec = [jnp.exp(a) for a in avec]
                for h in range(H):
                    hvec = jnp.full((_L,), h, dtype=jnp.int32)
                    plsc.store_scatter(pp_v, [j * _L + lane, hvec], pvec[h])
                for i16 in range(_L):
                    i = j * _L + i16
                    for h in range(H):
                        wv = jnp.full((_L,), pvec[h][i16])
                        for g in range(C // _L):
                            col = h * C + g * _L
                            xlr_v[i, pl.ds(col, _L)] = (
                                xlr_v[i, pl.ds(col, _L)] * wv)
                return cy

            lax.fori_loop(0, _CH // _L, grp, 0)

            pltpu.sync_copy(xlr_v, num_sh.at[di_v], add=True)
            pltpu.sync_copy(pp_v, den_sh.at[di_v], add=True)
            return carry

        lax.fori_loop(0, NCH, chunk, 0)
        plsc.subcore_barrier()

        pltpu.sync_copy(num_sh.at[pl.ds(s * RPT, RPT)],
                        nump_hbm.at[c, pl.ds(s * RPT, RPT)])
        pltpu.sync_copy(den_sh.at[pl.ds(s * RPT, RPT)],
                        denp_hbm.at[c, pl.ds(s * RPT, RPT)])

    return k(xl, xr, src, dst, attf, zrow, zden)


_NB = 2048


def _lin2_body(x_ref, wa_ref, wb_ref, oa_ref, ob_ref):
    x = x_ref[...]
    oa_ref[...] = jnp.dot(x, wa_ref[...], preferred_element_type=jnp.float32)
    ob_ref[...] = jnp.dot(x, wb_ref[...], preferred_element_type=jnp.float32)


def _lin2(x, Wa, Wb):
    N, K = x.shape
    Ma, Mb = Wa.shape[1], Wb.shape[1]
    return pl.pallas_call(
        _lin2_body,
        grid=(N // _NB,),
        in_specs=[
            pl.BlockSpec((_NB, K), lambda i: (i, 0)),
            pl.BlockSpec((K, Ma), lambda i: (0, 0)),
            pl.BlockSpec((K, Mb), lambda i: (0, 0)),
        ],
        out_specs=[
            pl.BlockSpec((_NB, Ma), lambda i: (i, 0)),
            pl.BlockSpec((_NB, Mb), lambda i: (i, 0)),
        ],
        out_shape=[
            jax.ShapeDtypeStruct((N, Ma), jnp.float32),
            jax.ShapeDtypeStruct((N, Mb), jnp.float32),
        ],
    )(x, Wa, Wb)


def _gat_norm(np_ref, dp_ref, xl_ref, xr_ref, A_ref, B_ref, b_ref, g_ref,
              be_ref):
    xl = xl_ref[...]
    t = xl + xr_ref[...]
    t = jnp.maximum(t, 0.2 * t)
    hp = lambda a, b: jnp.dot(a, b, preferred_element_type=jnp.float32,
                              precision=lax.Precision.HIGHEST)
    aself = hp(t, A_ref[...])
    sng = jnp.exp(aself)
    B = B_ref[...]
    H = sng.shape[1]
    den = dp_ref[0, :, :H] + dp_ref[1, :, :H] + sng
    num = np_ref[0] + np_ref[1] + hp(sng, B) * xl
    h = num * hp(1.0 / den, B)
    h = h + b_ref[...]
    mu = jnp.mean(h, axis=1, keepdims=True)
    var = jnp.mean((h - mu) ** 2, axis=1, keepdims=True)
    h = (h - mu) / jnp.sqrt(var + 1e-5) * g_ref[...] + be_ref[...]
    return jnp.maximum(h, 0.0)


def _combine0_body(np_ref, dp_ref, xl_ref, xr_ref, A_ref, B_ref, b_ref,
                   g_ref, be_ref, wl_ref, wr_ref, o1_ref, o2_ref):
    hr = _gat_norm(np_ref, dp_ref, xl_ref, xr_ref, A_ref, B_ref, b_ref,
                   g_ref, be_ref)
    o1_ref[...] = jnp.dot(hr, wl_ref[...], preferred_element_type=jnp.float32)
    o2_ref[...] = jnp.dot(hr, wr_ref[...], preferred_element_type=jnp.float32)


def _combine0(nump, denp, xl, xr, A, B, b, g, be, Wl, Wr):
    N, D = xl.shape
    H = A.shape[1]
    M = Wl.shape[1]
    full = lambda shape: pl.BlockSpec(shape, lambda i: tuple(0 for _ in shape))
    return pl.pallas_call(
        _combine0_body,
        grid=(N // _NB,),
        in_specs=[
            pl.BlockSpec((_NC, _NB, D), lambda i: (0, i, 0)),
            pl.BlockSpec((_NC, _NB, _L), lambda i: (0, i, 0)),
            pl.BlockSpec((_NB, D), lambda i: (i, 0)),
            pl.BlockSpec((_NB, D), lambda i: (i, 0)),
            full((D, H)),
            full((H, D)),
            full((1, D)),
            full((1, D)),
            full((1, D)),
            full((D, M)),
            full((D, M)),
        ],
        out_specs=[
            pl.BlockSpec((_NB, M), lambda i: (i, 0)),
            pl.BlockSpec((_NB, M), lambda i: (i, 0)),
        ],
        out_shape=[
            jax.ShapeDtypeStruct((N, M), jnp.float32),
            jax.ShapeDtypeStruct((N, M), jnp.float32),
        ],
    )(nump, denp, xl, xr, A, B, b, g, be, Wl, Wr)


def _combine1_body(np_ref, dp_ref, xl_ref, xr_ref, A_ref, B_ref, b_ref,
                   g_ref, be_ref, w1_ref, b1_ref, w2_ref, b2_ref, o_ref):
    hr = _gat_norm(np_ref, dp_ref, xl_ref, xr_ref, A_ref, B_ref, b_ref,
                   g_ref, be_ref)
    r = jnp.dot(hr, w1_ref[...], preferred_element_type=jnp.float32)
    r = jnp.maximum(r + b1_ref[...], 0.0)
    o_ref[...] = (jnp.dot(r, w2_ref[...], preferred_element_type=jnp.float32)
                  + b2_ref[...])


def _combine1(nump, denp, xl, xr, A, B, b, g, be, Wc1, bc1, Wc2, bc2):
    N, D = xl.shape
    H = A.shape[1]
    full = lambda shape: pl.BlockSpec(shape, lambda i: tuple(0 for _ in shape))
    return pl.pallas_call(
        _combine1_body,
        grid=(N // _NB,),
        in_specs=[
            pl.BlockSpec((_NC, _NB, D), lambda i: (0, i, 0)),
            pl.BlockSpec((_NC, _NB, _L), lambda i: (0, i, 0)),
            pl.BlockSpec((_NB, D), lambda i: (i, 0)),
            pl.BlockSpec((_NB, D), lambda i: (i, 0)),
            full((D, H)),
            full((H, D)),
            full((1, D)),
            full((1, D)),
            full((1, D)),
            full((D, D)),
            full((1, D)),
            full((D, 1)),
            full((1, 1)),
        ],
        out_specs=[pl.BlockSpec((_NB, 1), lambda i: (i, 0))],
        out_shape=[jax.ShapeDtypeStruct((N, 1), jnp.float32)],
    )(nump, denp, xl, xr, A, B, b, g, be, Wc1, bc1, Wc2, bc2)[0]


def kernel(x, edge_index, Wl0, Wr0, att0, b0, g0, be0, Wl1, Wr1, att1, b1,
           g1, be1, Wc1, bc1, Wc2, bc2):
    N = x.shape[0]
    src = edge_index[0].astype(jnp.int32)
    dst = edge_index[1].astype(jnp.int32)

    f32 = jnp.float32

    NP = -(-N // _NB) * _NB
    if NP != N:
        x = jnp.concatenate(
            [x, jnp.zeros((NP - N, x.shape[1]), x.dtype)], axis=0)

    H0, C0 = att0.shape
    D0 = H0 * C0
    xl0, xr0 = _lin2(x, Wl0, Wr0)
    att0f = att0.reshape(-1).astype(f32)
    nump0, denp0 = _edge_pass(xl0, xr0, src, dst, att0f, H0, C0)
    heads0 = jnp.arange(D0, dtype=jnp.int32) // C0
    B0 = (heads0[None, :] == jnp.arange(H0, dtype=jnp.int32)[:, None]).astype(f32)
    A0 = B0.T * att0f[:, None]
    xl1, xr1 = _combine0(nump0, denp0, xl0, xr0, A0, B0, b0.reshape(1, -1),
                         g0.reshape(1, -1), be0.reshape(1, -1), Wl1, Wr1)

    H1, C1 = att1.shape
    att1f = att1.reshape(-1).astype(f32)
    nump1, denp1 = _edge_pass(xl1, xr1, src, dst, att1f, H1, C1)
    A1 = att1f[:, None]
    B1 = jnp.ones((H1, H1 * C1), f32)
    out = _combine1(nump1, denp1, xl1, xr1, A1, B1, b1.reshape(1, -1),
                    g1.reshape(1, -1), be1.reshape(1, -1),
                    Wc1, bc1.reshape(1, -1), Wc2, bc2.reshape(1, -1))
    return out[:N]


# speedup vs baseline: 50.8794x; 1.1135x over previous
import functools

import jax
import jax.numpy as jnp
from jax import lax
from jax.experimental import pallas as pl
from jax.experimental.pallas import tpu as pltpu
from jax.experimental.pallas import tpu_sc as plsc

_NC = 2
_NS = 16
_L = 16
_CH = 80


def _edge_pass(xl, xr, src, dst, attf, H, C):
    N = xl.shape[0]
    E = src.shape[0]
    D = H * C
    NW = _NC * _NS
    EPW = E // NW
    NCH = EPW // _CH
    RPT = N // _NS
    assert EPW * NW == E and NCH * _CH == EPW and RPT * _NS == N

    zrow = jnp.zeros((RPT, D), jnp.float32)
    zden = jnp.zeros((RPT, _L), jnp.float32)

    mesh = plsc.VectorSubcoreMesh(core_axis_name="c", subcore_axis_name="s")

    @functools.partial(
        pl.kernel,
        out_type=(
            jax.ShapeDtypeStruct((_NC, N, D), jnp.float32),
            jax.ShapeDtypeStruct((_NC, N, _L), jnp.float32),
        ),
        mesh=mesh,
        compiler_params=pltpu.CompilerParams(
            needs_layout_passes=False, use_tc_tiling_on_sc=False),
        scratch_types=[
            pltpu.VMEM((D,), jnp.float32),
            pltpu.VMEM((_CH,), jnp.int32),
            pltpu.VMEM((_CH,), jnp.int32),
            pltpu.VMEM((_CH,), jnp.int32),
            pltpu.VMEM((_CH, D), jnp.float32),
            pltpu.VMEM((_CH, D), jnp.float32),
            pltpu.VMEM((_CH, D), jnp.float32),
            pltpu.VMEM((_CH, _L), jnp.float32),
            pltpu.VMEM((_CH, _L), jnp.float32),
            pltpu.VMEM_SHARED((N, D), jnp.float32),
            pltpu.VMEM_SHARED((N, _L), jnp.float32),
            pltpu.SemaphoreType.DMA,
            pltpu.SemaphoreType.DMA,
            pltpu.SemaphoreType.DMA,
        ],
    )
    def k(xl_hbm, xr_hbm, src_hbm, dst_hbm, att_hbm, zrow_hbm, zden_hbm,
          nump_hbm, denp_hbm,
          att_v, si_v, di0, di1, xlr0, xlr1, xrr_v, pp0, pp1,
          num_sh, den_sh, gsem, ssem0, ssem1):
        c = lax.axis_index("c")
        s = lax.axis_index("s")
        wid = s * _NC + c

        di = [di0, di1]
        xlr = [xlr0, xlr1]
        pp = [pp0, pp1]
        ssem = [ssem0, ssem1]

        pltpu.sync_copy(att_hbm, att_v)
        pltpu.sync_copy(zrow_hbm, num_sh.at[pl.ds(s * RPT, RPT)])
        pltpu.sync_copy(zden_hbm, den_sh.at[pl.ds(s * RPT, RPT)])

        for sl in range(2):
            def zi(i, cy, sl=sl):
                pp[sl][i, pl.ds(0, _L)] = jnp.zeros((_L,), jnp.float32)
                return cy

            lax.fori_loop(0, _CH, zi, 0)
        plsc.subcore_barrier()

        attv = [att_v[pl.ds(g * _L, _L)] for g in range(D // _L)]
        lane = lax.iota(jnp.int32, _L)
        lmask = [lane == i for i in range(_L)]
        base0 = wid * EPW

        def wait_out(sl):
            pltpu.make_async_copy(xlr[sl], num_sh.at[di[sl]],
                                  ssem[sl]).wait()
            pltpu.make_async_copy(pp[sl], den_sh.at[di[sl]],
                                  ssem[sl]).wait()

        def chunk_body(kk, sl):
            @pl.when(kk >= 2)
            def _():
                wait_out(sl)

            base = base0 + kk * _CH
            pltpu.sync_copy(src_hbm.at[pl.ds(base, _CH)], si_v)
            pltpu.sync_copy(dst_hbm.at[pl.ds(base, _CH)], di[sl])
            cp1 = pltpu.async_copy(xl_hbm.at[si_v], xlr[sl], gsem)
            cp2 = pltpu.async_copy(xr_hbm.at[di[sl]], xrr_v, gsem)
            cp1.wait()
            cp2.wait()

            xl_v = xlr[sl]
            pp_v = pp[sl]

            def grp(j, cy):
                avec = [jnp.zeros((_L,), jnp.float32) for _ in range(H)]
                for i16 in range(_L):
                    i = j * _L + i16
                    for h in range(H):
                        acc = None
                        for g in range(C // _L):
                            col = h * C + g * _L
                            v = (xl_v[i, pl.ds(col, _L)]
                                 + xrr_v[i, pl.ds(col, _L)])
                            t = jnp.maximum(v, 0.2 * v)
                            term = t * attv[h * (C // _L) + g]
                            acc = term if acc is None else acc + term
                        avec[h] = jnp.where(lmask[i16], jnp.sum(acc), avec[h])
                pvec = [jnp.exp(a) for a in avec]
                for h in range(H):
                    hvec = jnp.full((_L,), h, dtype=jnp.int32)
                    plsc.store_scatter(pp_v, [j * _L + lane, hvec], pvec[h])
                for i16 in range(_L):
                    i = j * _L + i16
                    for h in range(H):
                        wv = jnp.full((_L,), pvec[h][i16])
                        for g in range(C // _L):
                            col = h * C + g * _L
                            xl_v[i, pl.ds(col, _L)] = (
                                xl_v[i, pl.ds(col, _L)] * wv)
                return cy

            lax.fori_loop(0, _CH // _L, grp, 0)

            pltpu.async_copy(xlr[sl], num_sh.at[di[sl]], ssem[sl], add=True)
            pltpu.async_copy(pp[sl], den_sh.at[di[sl]], ssem[sl], add=True)

        def pair(q, carry):
            chunk_body(2 * q, 0)
            chunk_body(2 * q + 1, 1)
            return carry

        lax.fori_loop(0, NCH // 2, pair, 0)
        if NCH % 2:
            chunk_body(NCH - 1, 0)
        wait_out(0)
        if NCH >= 2:
            wait_out(1)
        plsc.subcore_barrier()

        pltpu.sync_copy(num_sh.at[pl.ds(s * RPT, RPT)],
                        nump_hbm.at[c, pl.ds(s * RPT, RPT)])
        pltpu.sync_copy(den_sh.at[pl.ds(s * RPT, RPT)],
                        denp_hbm.at[c, pl.ds(s * RPT, RPT)])

    return k(xl, xr, src, dst, attf, zrow, zden)


_NB = 2048


def _lin2_body(x_ref, wa_ref, wb_ref, oa_ref, ob_ref):
    x = x_ref[...]
    oa_ref[...] = jnp.dot(x, wa_ref[...], preferred_element_type=jnp.float32)
    ob_ref[...] = jnp.dot(x, wb_ref[...], preferred_element_type=jnp.float32)


def _lin2(x, Wa, Wb):
    N, K = x.shape
    Ma, Mb = Wa.shape[1], Wb.shape[1]
    return pl.pallas_call(
        _lin2_body,
        grid=(N // _NB,),
        in_specs=[
            pl.BlockSpec((_NB, K), lambda i: (i, 0)),
            pl.BlockSpec((K, Ma), lambda i: (0, 0)),
            pl.BlockSpec((K, Mb), lambda i: (0, 0)),
        ],
        out_specs=[
            pl.BlockSpec((_NB, Ma), lambda i: (i, 0)),
            pl.BlockSpec((_NB, Mb), lambda i: (i, 0)),
        ],
        out_shape=[
            jax.ShapeDtypeStruct((N, Ma), jnp.float32),
            jax.ShapeDtypeStruct((N, Mb), jnp.float32),
        ],
    )(x, Wa, Wb)


def _gat_norm(np_ref, dp_ref, xl_ref, xr_ref, A_ref, B_ref, b_ref, g_ref,
              be_ref):
    xl = xl_ref[...]
    t = xl + xr_ref[...]
    t = jnp.maximum(t, 0.2 * t)
    hp = lambda a, b: jnp.dot(a, b, preferred_element_type=jnp.float32,
                              precision=lax.Precision.HIGHEST)
    aself = hp(t, A_ref[...])
    sng = jnp.exp(aself)
    B = B_ref[...]
    H = sng.shape[1]
    den = dp_ref[0, :, :H] + dp_ref[1, :, :H] + sng
    num = np_ref[0] + np_ref[1] + hp(sng, B) * xl
    h = num * hp(1.0 / den, B)
    h = h + b_ref[...]
    mu = jnp.mean(h, axis=1, keepdims=True)
    var = jnp.mean((h - mu) ** 2, axis=1, keepdims=True)
    h = (h - mu) / jnp.sqrt(var + 1e-5) * g_ref[...] + be_ref[...]
    return jnp.maximum(h, 0.0)


def _combine0_body(np_ref, dp_ref, xl_ref, xr_ref, A_ref, B_ref, b_ref,
                   g_ref, be_ref, wl_ref, wr_ref, o1_ref, o2_ref):
    hr = _gat_norm(np_ref, dp_ref, xl_ref, xr_ref, A_ref, B_ref, b_ref,
                   g_ref, be_ref)
    o1_ref[...] = jnp.dot(hr, wl_ref[...], preferred_element_type=jnp.float32)
    o2_ref[...] = jnp.dot(hr, wr_ref[...], preferred_element_type=jnp.float32)


def _combine0(nump, denp, xl, xr, A, B, b, g, be, Wl, Wr):
    N, D = xl.shape
    H = A.shape[1]
    M = Wl.shape[1]
    full = lambda shape: pl.BlockSpec(shape, lambda i: tuple(0 for _ in shape))
    return pl.pallas_call(
        _combine0_body,
        grid=(N // _NB,),
        in_specs=[
            pl.BlockSpec((_NC, _NB, D), lambda i: (0, i, 0)),
            pl.BlockSpec((_NC, _NB, _L), lambda i: (0, i, 0)),
            pl.BlockSpec((_NB, D), lambda i: (i, 0)),
            pl.BlockSpec((_NB, D), lambda i: (i, 0)),
            full((D, H)),
            full((H, D)),
            full((1, D)),
            full((1, D)),
            full((1, D)),
            full((D, M)),
            full((D, M)),
        ],
        out_specs=[
            pl.BlockSpec((_NB, M), lambda i: (i, 0)),
            pl.BlockSpec((_NB, M), lambda i: (i, 0)),
        ],
        out_shape=[
            jax.ShapeDtypeStruct((N, M), jnp.float32),
            jax.ShapeDtypeStruct((N, M), jnp.float32),
        ],
    )(nump, denp, xl, xr, A, B, b, g, be, Wl, Wr)


def _combine1_body(np_ref, dp_ref, xl_ref, xr_ref, A_ref, B_ref, b_ref,
                   g_ref, be_ref, w1_ref, b1_ref, w2_ref, b2_ref, o_ref):
    hr = _gat_norm(np_ref, dp_ref, xl_ref, xr_ref, A_ref, B_ref, b_ref,
                   g_ref, be_ref)
    r = jnp.dot(hr, w1_ref[...], preferred_element_type=jnp.float32)
    r = jnp.maximum(r + b1_ref[...], 0.0)
    o_ref[...] = (jnp.dot(r, w2_ref[...], preferred_element_type=jnp.float32)
                  + b2_ref[...])


def _combine1(nump, denp, xl, xr, A, B, b, g, be, Wc1, bc1, Wc2, bc2):
    N, D = xl.shape
    H = A.shape[1]
    full = lambda shape: pl.BlockSpec(shape, lambda i: tuple(0 for _ in shape))
    return pl.pallas_call(
        _combine1_body,
        grid=(N // _NB,),
        in_specs=[
            pl.BlockSpec((_NC, _NB, D), lambda i: (0, i, 0)),
            pl.BlockSpec((_NC, _NB, _L), lambda i: (0, i, 0)),
            pl.BlockSpec((_NB, D), lambda i: (i, 0)),
            pl.BlockSpec((_NB, D), lambda i: (i, 0)),
            full((D, H)),
            full((H, D)),
            full((1, D)),
            full((1, D)),
            full((1, D)),
            full((D, D)),
            full((1, D)),
            full((D, 1)),
            full((1, 1)),
        ],
        out_specs=[pl.BlockSpec((_NB, 1), lambda i: (i, 0))],
        out_shape=[jax.ShapeDtypeStruct((N, 1), jnp.float32)],
    )(nump, denp, xl, xr, A, B, b, g, be, Wc1, bc1, Wc2, bc2)[0]


def kernel(x, edge_index, Wl0, Wr0, att0, b0, g0, be0, Wl1, Wr1, att1, b1,
           g1, be1, Wc1, bc1, Wc2, bc2):
    N = x.shape[0]
    src = edge_index[0].astype(jnp.int32)
    dst = edge_index[1].astype(jnp.int32)

    f32 = jnp.float32

    NP = -(-N // _NB) * _NB
    if NP != N:
        x = jnp.concatenate(
            [x, jnp.zeros((NP - N, x.shape[1]), x.dtype)], axis=0)

    H0, C0 = att0.shape
    D0 = H0 * C0
    xl0, xr0 = _lin2(x, Wl0, Wr0)
    att0f = att0.reshape(-1).astype(f32)
    nump0, denp0 = _edge_pass(xl0, xr0, src, dst, att0f, H0, C0)
    heads0 = jnp.arange(D0, dtype=jnp.int32) // C0
    B0 = (heads0[None, :] == jnp.arange(H0, dtype=jnp.int32)[:, None]).astype(f32)
    A0 = B0.T * att0f[:, None]
    xl1, xr1 = _combine0(nump0, denp0, xl0, xr0, A0, B0, b0.reshape(1, -1),
                         g0.reshape(1, -1), be0.reshape(1, -1), Wl1, Wr1)

    H1, C1 = att1.shape
    att1f = att1.reshape(-1).astype(f32)
    nump1, denp1 = _edge_pass(xl1, xr1, src, dst, att1f, H1, C1)
    A1 = att1f[:, None]
    B1 = jnp.ones((H1, H1 * C1), f32)
    out = _combine1(nump1, denp1, xl1, xr1, A1, B1, b1.reshape(1, -1),
                    g1.reshape(1, -1), be1.reshape(1, -1),
                    Wc1, bc1.reshape(1, -1), Wc2, bc2.reshape(1, -1))
    return out[:N]
